# baseline (speedup 1.0000x reference)
import jax
import jax.numpy as jnp
from jax import lax
from jax.experimental import pallas as pl
from jax.experimental.pallas import tpu as pltpu

_EPS = 1e-9
_LN2 = 0.6931471805599453
_ILN2 = 1.4426950408889634
_ISQ = 1.2011224087864498
_C2 = _ILN2


def _rotr1(a):
    return jnp.concatenate([a[:, -1:], a[:, :-1]], axis=1)


def _rotl1(a):
    return jnp.concatenate([a[:, 1:], a[:, :1]], axis=1)


def _roll_static(a, s):
    n = a.shape[1]
    s %= n
    if s == 0:
        return a
    return jnp.concatenate([a[:, n - s:], a[:, :n - s]], axis=1)


def _cumsum_lanes(a, n):
    k = 1
    while k < n:
        shifted = jnp.concatenate(
            [jnp.zeros((a.shape[0], k), a.dtype), a[:, :-k]], axis=1
        )
        a = a + shifted
        k *= 2
    return a


def _trans(ag, a2, bs, b2):
    u = ag * bs
    one_minus_g = 0.5 * (1.0 + u * lax.rsqrt(u * u + _EPS))
    sm2 = jnp.minimum(a2, b2) - jnp.log2(1.0 + jnp.exp2(-jnp.abs(a2 - b2)))
    return _C2 + one_minus_g * sm2


def _softmin3(d1, d2, d3):
    m = jnp.minimum(d1, jnp.minimum(d2, d3))
    return m - jnp.log2(
        jnp.exp2(m - d1) + jnp.exp2(m - d2) + jnp.exp2(m - d3)
    )


def _step(xw, dxgw, dx2w, diag1, diag2, ybuf_p, ybuf, dyg=None, dy2=None):
    dxy = xw - ybuf
    match = dxy * dxy
    if dyg is None:
        dy = ybuf - ybuf_p
        dyg = dy * _LN2
        dy2 = dy * dy
    up = _trans(dxgw, dx2w, dxy, match)
    left = _trans(dyg, dy2, -dxy, match)
    d_diag = _rotr1(diag2) + match
    d_up = _rotr1(diag1) + up
    d_left = diag1 + left
    return _softmin3(d_diag, d_up, d_left)


def _run_phase(body_fn, d_lo, d_hi, carry, unroll=16):
    n_steps = d_hi - d_lo
    groups = n_steps // unroll

    def body_u(t, c):
        d = d_lo + unroll * t
        for i in range(unroll):
            c = body_fn(d + i, c)
        return c

    carry = lax.fori_loop(0, groups, body_u, carry)
    for d in range(d_lo + groups * unroll, d_hi):
        carry = body_fn(d, carry)
    return carry


def _msm_wavefront(x_ref, y_ref, yrev_ref, o_ref):
    xv = x_ref[...] * _ISQ
    yv = y_ref[...] * _ISQ
    yrev = yrev_ref[...] * _ISQ
    bb, n = xv.shape
    ph = min(128, n)
    nph = n // ph
    iota = lax.broadcasted_iota(jnp.int32, (bb, n), 1)
    iota0 = iota == 0

    x0 = xv[:, :1]
    y0 = yv[:, :1]
    c00 = (x0 - y0) ** 2

    dxv = xv - _rotr1(xv)
    dxg = dxv * _LN2
    dx2v = dxv * dxv

    bx = xv - y0
    tcol = jnp.where(iota0, 0.0, _trans(dxg, dx2v, bx, bx * bx))
    col0v = c00 + _cumsum_lanes(tcol, n)

    dyv = yv - _rotr1(yv)
    by = yv - x0
    trow = jnp.where(iota0, 0.0, _trans(dyv * _LN2, dyv * dyv, by, by * by))
    row0 = c00 + _cumsum_lanes(trow, n)

    dvr = yrev - _rotl1(yrev)
    dyg_rev = dvr * _LN2
    dy2_rev = dvr * dvr

    diag1_full = jnp.where(
        iota0, _rotl1(row0), jnp.where(iota == 1, col0v, 0.0)
    )
    diag1 = diag1_full[:, :ph]
    diag2 = jnp.where(iota0[:, :ph], c00, 0.0)

    for p in range(nph):
        w = ph * (p + 1)
        d_lo = max(2, ph * p)
        d_hi = ph * (p + 1)
        if p > 0:
            pad = jnp.zeros((bb, ph), jnp.float32)
            diag1 = jnp.concatenate([diag1, pad], axis=1)
            diag2 = jnp.concatenate([diag2, pad], axis=1)
        ybuf = _roll_static(yrev, d_lo)[:, :w]
        yfeed = _roll_static(yv, -(d_lo - 1))[:, :w]
        r0s = _roll_static(row0, -(d_lo - 1))[:, :w]
        xw = xv[:, :w]
        dxgw = dxg[:, :w]
        dx2w = dx2v[:, :w]
        col0w = col0v[:, :w]
        iw = iota[:, :w]
        i0w = iota0[:, :w]

        def body_a(d, carry, xw=xw, dxgw=dxgw, dx2w=dx2w, col0w=col0w,
                   iw=iw, i0w=i0w):
            diag1, diag2, ybuf_p, yfeed_p, r0s_p = carry
            yfeed = _rotl1(yfeed_p)
            r0s = _rotl1(r0s_p)
            ybuf = jnp.where(i0w, yfeed, _rotr1(ybuf_p))
            cur = _step(xw, dxgw, dx2w, diag1, diag2, ybuf_p, ybuf)
            cur = jnp.where(i0w, r0s, cur)
            cur = jnp.where(iw == d, col0w, cur)
            return (cur, diag1, ybuf, yfeed, r0s)

        diag1, diag2, ybuf, yfeed, r0s = _run_phase(
            body_a, d_lo, d_hi, (diag1, diag2, ybuf, yfeed, r0s)
        )

    for q in range(nph):
        s = ph * q
        d_lo = n + ph * q
        d_hi = min(n + ph * (q + 1), 2 * n - 1)
        if q > 0:
            diag1 = diag1[:, ph:]
            diag2 = diag2[:, ph:]
        ybuf = _roll_static(yrev, d_lo)[:, s:]
        dygb = _roll_static(dyg_rev, d_lo)[:, s:]
        dy2b = _roll_static(dy2_rev, d_lo)[:, s:]
        xw = xv[:, s:]
        dxgw = dxg[:, s:]
        dx2w = dx2v[:, s:]

        def body_b(d, carry, xw=xw, dxgw=dxgw, dx2w=dx2w):
            diag1, diag2, ybuf_p, dygb_p, dy2b_p = carry
            ybuf = _rotr1(ybuf_p)
            dygb = _rotr1(dygb_p)
            dy2b = _rotr1(dy2b_p)
            cur = _step(xw, dxgw, dx2w, diag1, diag2, ybuf_p, ybuf,
                        dygb, dy2b)
            return (cur, diag1, ybuf, dygb, dy2b)

        diag1, diag2, ybuf, dygb, dy2b = _run_phase(
            body_b, d_lo, d_hi, (diag1, diag2, ybuf, dygb, dy2b)
        )

    wf = diag1.shape[1]
    iota_f = lax.broadcasted_iota(jnp.int32, (bb, wf), 1)
    cost = jnp.sum(
        jnp.where(iota_f == wf - 1, diag1, 0.0), axis=1, keepdims=True
    ) * _LN2
    o_ref[...] = jnp.broadcast_to(cost, (bb, 128))


def _build_call(b, n, interpret=False):
    return pl.pallas_call(
        _msm_wavefront,
        out_shape=jax.ShapeDtypeStruct((b, 128), jnp.float32),
        interpret=interpret,
    )


def kernel(x, y):
    b, _, n = x.shape
    x2 = x[:, 0, :]
    y2 = y[:, 0, :]
    yrev = y2[:, ::-1]
    out = _build_call(b, n)(x2, y2, yrev)
    return out[:, 0].mean()


# speedup vs baseline: 1.0075x; 1.0075x over previous
import jax
import jax.numpy as jnp
from jax import lax
from jax.experimental import pallas as pl

_EPS = 1e-9
_LN2 = 0.6931471805599453
_ILN2 = 1.4426950408889634
_ISQ = 1.2011224087864498
_C2 = _ILN2


def _rotr1(a):
    return jnp.concatenate([a[:, -1:], a[:, :-1]], axis=1)


def _rotl1(a):
    return jnp.concatenate([a[:, 1:], a[:, :1]], axis=1)


def _roll_static(a, s):
    n = a.shape[1]
    s %= n
    if s == 0:
        return a
    return jnp.concatenate([a[:, n - s:], a[:, :n - s]], axis=1)


def _cumsum_lanes(a, n):
    k = 1
    while k < n:
        shifted = jnp.concatenate(
            [jnp.zeros((a.shape[0], k), a.dtype), a[:, :-k]], axis=1
        )
        a = a + shifted
        k *= 2
    return a


def _trans(ag, a2, bs, b2):
    u = ag * bs
    one_minus_g = 0.5 * (1.0 + u * lax.rsqrt(u * u + _EPS))
    sm2 = jnp.minimum(a2, b2) - jnp.log2(1.0 + jnp.exp2(-jnp.abs(a2 - b2)))
    return _C2 + one_minus_g * sm2


def _softmin3(d1, d2, d3):
    m = jnp.minimum(d1, jnp.minimum(d2, d3))
    return m - jnp.log2(
        jnp.exp2(m - d1) + jnp.exp2(m - d2) + jnp.exp2(m - d3)
    )


def _step(xw, dxgw, dx2w, diag1, diag2, ybuf_p, ybuf):
    dxy = xw - ybuf
    match = dxy * dxy
    dy = ybuf - ybuf_p
    up = _trans(dxgw, dx2w, dxy, match)
    left = _trans(dy * _LN2, dy * dy, -dxy, match)
    d_diag = _rotr1(diag2) + match
    d_up = _rotr1(diag1) + up
    d_left = diag1 + left
    return _softmin3(d_diag, d_up, d_left)


def _run_phase(body_fn, d_lo, d_hi, carry, unroll=16):
    n_steps = d_hi - d_lo
    groups = n_steps // unroll

    def body_u(t, c):
        d = d_lo + unroll * t
        for i in range(unroll):
            c = body_fn(d + i, c)
        return c

    carry = lax.fori_loop(0, groups, body_u, carry)
    for d in range(d_lo + groups * unroll, d_hi):
        carry = body_fn(d, carry)
    return carry


def _msm_wavefront(x_ref, y_ref, yrev_ref, o_ref):
    xv = x_ref[...] * _ISQ
    yv = y_ref[...] * _ISQ
    yrev = yrev_ref[...] * _ISQ
    bb, n = xv.shape
    ph = min(128, n)
    nph = n // ph
    iota = lax.broadcasted_iota(jnp.int32, (bb, n), 1)
    iota0 = iota == 0

    x0 = xv[:, :1]
    y0 = yv[:, :1]
    c00 = (x0 - y0) ** 2

    dxv = xv - _rotr1(xv)
    dxg = dxv * _LN2
    dx2v = dxv * dxv

    bx = xv - y0
    tcol = jnp.where(iota0, 0.0, _trans(dxg, dx2v, bx, bx * bx))
    col0v = c00 + _cumsum_lanes(tcol, n)

    dyv = yv - _rotr1(yv)
    by = yv - x0
    trow = jnp.where(iota0, 0.0, _trans(dyv * _LN2, dyv * dyv, by, by * by))
    row0 = c00 + _cumsum_lanes(trow, n)

    diag1_full = jnp.where(
        iota0, _rotl1(row0), jnp.where(iota == 1, col0v, 0.0)
    )
    diag1 = diag1_full[:, :ph]
    diag2 = jnp.where(iota0[:, :ph], c00, 0.0)

    for p in range(nph):
        w = ph * (p + 1)
        d_lo = max(2, ph * p)
        d_hi = ph * (p + 1)
        if p > 0:
            pad = jnp.zeros((bb, ph), jnp.float32)
            diag1 = jnp.concatenate([diag1, pad], axis=1)
            diag2 = jnp.concatenate([diag2, pad], axis=1)
        ybuf = _roll_static(yrev, d_lo)[:, :w]
        yfeed = _roll_static(yv, -(d_lo - 1))[:, :w]
        r0s = _roll_static(row0, -(d_lo - 1))[:, :w]
        xw = xv[:, :w]
        dxgw = dxg[:, :w]
        dx2w = dx2v[:, :w]
        col0w = col0v[:, :w]
        iw = iota[:, :w]
        i0w = iota0[:, :w]

        def body_a(d, carry, xw=xw, dxgw=dxgw, dx2w=dx2w, col0w=col0w,
                   iw=iw, i0w=i0w):
            diag1, diag2, ybuf_p, yfeed_p, r0s_p = carry
            yfeed = _rotl1(yfeed_p)
            r0s = _rotl1(r0s_p)
            ybuf = jnp.where(i0w, yfeed, _rotr1(ybuf_p))
            cur = _step(xw, dxgw, dx2w, diag1, diag2, ybuf_p, ybuf)
            cur = jnp.where(i0w, r0s, cur)
            cur = jnp.where(iw == d, col0w, cur)
            return (cur, diag1, ybuf, yfeed, r0s)

        diag1, diag2, ybuf, yfeed, r0s = _run_phase(
            body_a, d_lo, d_hi, (diag1, diag2, ybuf, yfeed, r0s)
        )

    for q in range(nph):
        s = ph * q
        d_lo = n + ph * q
        d_hi = min(n + ph * (q + 1), 2 * n - 1)
        if q > 0:
            diag1 = diag1[:, ph:]
            diag2 = diag2[:, ph:]
        ybuf = _roll_static(yrev, d_lo)[:, s:]
        xw = xv[:, s:]
        dxgw = dxg[:, s:]
        dx2w = dx2v[:, s:]

        def body_b(d, carry, xw=xw, dxgw=dxgw, dx2w=dx2w):
            diag1, diag2, ybuf_p = carry
            ybuf = _rotr1(ybuf_p)
            cur = _step(xw, dxgw, dx2w, diag1, diag2, ybuf_p, ybuf)
            return (cur, diag1, ybuf)

        diag1, diag2, ybuf = _run_phase(
            body_b, d_lo, d_hi, (diag1, diag2, ybuf)
        )

    wf = diag1.shape[1]
    iota_f = lax.broadcasted_iota(jnp.int32, (bb, wf), 1)
    cost = jnp.sum(
        jnp.where(iota_f == wf - 1, diag1, 0.0), axis=1, keepdims=True
    ) * _LN2
    o_ref[...] = jnp.broadcast_to(cost, (bb, 128))


def _build_call(b, n, interpret=False):
    return pl.pallas_call(
        _msm_wavefront,
        out_shape=jax.ShapeDtypeStruct((b, 128), jnp.float32),
        interpret=interpret,
    )


def kernel(x, y):
    b, _, n = x.shape
    x2 = x[:, 0, :]
    y2 = y[:, 0, :]
    yrev = y2[:, ::-1]
    out = _build_call(b, n)(x2, y2, yrev)
    return out[:, 0].mean()


# speedup vs baseline: 1.0077x; 1.0002x over previous
import jax
import jax.numpy as jnp
from jax import lax
from jax.experimental import pallas as pl

_EPS = 1e-9
_LN2 = 0.6931471805599453
_ILN2 = 1.4426950408889634
_ISQ = 1.2011224087864498
_C2 = _ILN2


def _rotr1(a):
    return jnp.concatenate([a[:, -1:], a[:, :-1]], axis=1)


def _rotl1(a):
    return jnp.concatenate([a[:, 1:], a[:, :1]], axis=1)


def _roll_static(a, s):
    n = a.shape[1]
    s %= n
    if s == 0:
        return a
    return jnp.concatenate([a[:, n - s:], a[:, :n - s]], axis=1)


def _cumsum_lanes(a, n):
    k = 1
    while k < n:
        shifted = jnp.concatenate(
            [jnp.zeros((a.shape[0], k), a.dtype), a[:, :-k]], axis=1
        )
        a = a + shifted
        k *= 2
    return a


def _trans(ag, a2, bs, b2):
    u = ag * bs
    one_minus_g = 0.5 * (1.0 + u * lax.rsqrt(u * u + _EPS))
    sm2 = jnp.minimum(a2, b2) - jnp.log2(1.0 + jnp.exp2(-jnp.abs(a2 - b2)))
    return _C2 + one_minus_g * sm2


def _softmin3(d1, d2, d3):
    m = jnp.minimum(d1, jnp.minimum(d2, d3))
    return m - jnp.log2(
        jnp.exp2(m - d1) + jnp.exp2(m - d2) + jnp.exp2(m - d3)
    )


def _step(xw, dxgw, dx2w, diag1, diag2, ybuf_p, ybuf):
    dxy = xw - ybuf
    match = dxy * dxy
    dy = ybuf - ybuf_p
    up = _trans(dxgw, dx2w, dxy, match)
    left = _trans(dy * _LN2, dy * dy, -dxy, match)
    d_diag = _rotr1(diag2) + match
    d_up = _rotr1(diag1) + up
    d_left = diag1 + left
    return _softmin3(d_diag, d_up, d_left)


def _run_phase(body_fn, d_lo, d_hi, carry, unroll=16):
    n_steps = d_hi - d_lo
    groups = n_steps // unroll

    def body_u(t, c):
        d = d_lo + unroll * t
        for i in range(unroll):
            c = body_fn(d + i, c)
        return c

    carry = lax.fori_loop(0, groups, body_u, carry)
    for d in range(d_lo + groups * unroll, d_hi):
        carry = body_fn(d, carry)
    return carry


def _msm_wavefront(x_ref, y_ref, yrev_ref, o_ref):
    xv = x_ref[...] * _ISQ
    yv = y_ref[...] * _ISQ
    yrev = yrev_ref[...] * _ISQ
    bb, n = xv.shape
    ph = min(128, n)
    nph = n // ph
    iota = lax.broadcasted_iota(jnp.int32, (bb, n), 1)
    iota0 = iota == 0

    x0 = xv[:, :1]
    y0 = yv[:, :1]
    c00 = (x0 - y0) ** 2

    dxv = xv - _rotr1(xv)
    dxg = dxv * _LN2
    dx2v = dxv * dxv

    bx = xv - y0
    tcol = jnp.where(iota0, 0.0, _trans(dxg, dx2v, bx, bx * bx))
    col0v = c00 + _cumsum_lanes(tcol, n)

    dyv = yv - _rotr1(yv)
    by = yv - x0
    trow = jnp.where(iota0, 0.0, _trans(dyv * _LN2, dyv * dyv, by, by * by))
    row0 = c00 + _cumsum_lanes(trow, n)

    diag1_full = jnp.where(
        iota0, _rotl1(row0), jnp.where(iota == 1, col0v, 0.0)
    )
    diag1 = diag1_full[:, :ph]
    diag2 = jnp.where(iota0[:, :ph], c00, 0.0)

    for p in range(nph):
        w = ph * (p + 1)
        d_lo = max(2, ph * p)
        d_hi = ph * (p + 1)
        if p > 0:
            pad = jnp.zeros((bb, ph), jnp.float32)
            diag1 = jnp.concatenate([diag1, pad], axis=1)
            diag2 = jnp.concatenate([diag2, pad], axis=1)
        ybuf = _roll_static(yrev, d_lo)[:, :w]
        yfeed = _roll_static(yv, -(d_lo - 1))[:, :w]
        r0s = _roll_static(row0, -(d_lo - 1))[:, :w]
        xw = xv[:, :w]
        dxgw = dxg[:, :w]
        dx2w = dx2v[:, :w]
        col0w = col0v[:, :w]
        iw = iota[:, :w]
        i0w = iota0[:, :w]

        def body_a(d, carry, xw=xw, dxgw=dxgw, dx2w=dx2w, col0w=col0w,
                   iw=iw, i0w=i0w):
            diag1, diag2, ybuf_p, yfeed_p, r0s_p = carry
            yfeed = _rotl1(yfeed_p)
            r0s = _rotl1(r0s_p)
            ybuf = jnp.where(i0w, yfeed, _rotr1(ybuf_p))
            cur = _step(xw, dxgw, dx2w, diag1, diag2, ybuf_p, ybuf)
            cur = jnp.where(i0w, r0s, cur)
            cur = jnp.where(iw == d, col0w, cur)
            return (cur, diag1, ybuf, yfeed, r0s)

        diag1, diag2, ybuf, yfeed, r0s = _run_phase(
            body_a, d_lo, d_hi, (diag1, diag2, ybuf, yfeed, r0s)
        )

    for q in range(nph):
        s = ph * q
        d_lo = n + ph * q
        d_hi = min(n + ph * (q + 1), 2 * n - 1)
        if q > 0:
            diag1 = diag1[:, ph:]
            diag2 = diag2[:, ph:]
        ybuf = _roll_static(yrev, d_lo)[:, s:]
        xw = xv[:, s:]
        dxgw = dxg[:, s:]
        dx2w = dx2v[:, s:]

        def body_b(d, carry, xw=xw, dxgw=dxgw, dx2w=dx2w):
            diag1, diag2, ybuf_p = carry
            ybuf = _rotr1(ybuf_p)
            cur = _step(xw, dxgw, dx2w, diag1, diag2, ybuf_p, ybuf)
            return (cur, diag1, ybuf)

        diag1, diag2, ybuf = _run_phase(
            body_b, d_lo, d_hi, (diag1, diag2, ybuf)
        )

    wf = diag1.shape[1]
    iota_f = lax.broadcasted_iota(jnp.int32, (bb, wf), 1)
    cost = jnp.sum(
        jnp.where(iota_f == wf - 1, diag1, 0.0), axis=1, keepdims=True
    ) * _LN2
    o_ref[...] = jnp.broadcast_to(cost, (bb, 128))


def kernel(x, y):
    b, _, n = x.shape
    x2 = x[:, 0, :]
    y2 = y[:, 0, :]
    yrev = y2[:, ::-1]
    out = pl.pallas_call(
        _msm_wavefront,
        out_shape=jax.ShapeDtypeStruct((b, 128), jnp.float32),
    )(x2, y2, yrev)
    return out[:, 0].mean()
